# baseline (speedup 1.0000x reference)
import functools

import jax
import jax.numpy as jnp
from jax import lax
from jax.experimental import pallas as pl
from jax.experimental.pallas import tpu as pltpu
from jax.experimental.pallas import tpu_sc as plsc

N_NODES = 100000
EMB_DIM = 64

NUM_CORES = 2
NUM_SUBCORES = 16
NW = NUM_CORES * NUM_SUBCORES

BLK = 128
NFULL = N_NODES // BLK
TAIL = N_NODES - NFULL * BLK
NPAIR = N_NODES // 2
N_PAD = (NFULL + 1) * BLK
NBLK_B = N_PAD // BLK
L = 16

_mesh = plsc.VectorSubcoreMesh(core_axis_name="c", subcore_axis_name="s")
_params = pltpu.CompilerParams(needs_layout_passes=False)
_params_b = pltpu.CompilerParams(
    needs_layout_passes=False, disable_bounds_checks=True
)


def _iota():
    return lax.iota(jnp.int32, L)


@functools.partial(
    pl.kernel,
    mesh=_mesh,
    compiler_params=_params,
    out_type=jax.ShapeDtypeStruct((NPAIR, 128), jnp.float32),
    scratch_types=[
        pltpu.VMEM((2, 64, BLK), jnp.float32),
        pltpu.VMEM((2, 64, 129), jnp.float32),
        pltpu.SemaphoreType.DMA,
        pltpu.SemaphoreType.DMA,
        pltpu.SemaphoreType.DMA,
        pltpu.SemaphoreType.DMA,
    ],
)
def _pairs_kernel(tabt_hbm, tail_hbm, pairs_hbm, tbuf, pbuf, s0, s1, w0, w1):
    wid = lax.axis_index("s") * NUM_CORES + lax.axis_index("c")
    it = _iota()
    rowv = [(it + k * 16) // 2 for k in range(8)]
    par64 = (it % 2) * 64
    sems = [s0, s1]
    wsems = [w0, w1]

    def fire(b, buf):
        @pl.when(b < NFULL)
        def _():
            pltpu.async_copy(
                tabt_hbm.at[:, pl.ds(b * BLK, BLK)], tbuf.at[buf], sems[buf]
            )

    def drain(buf):
        pltpu.make_async_copy(
            tabt_hbm.at[:, pl.ds(0, BLK)], tbuf.at[buf], sems[buf]
        ).wait()

    def drain_w(buf):
        pltpu.make_async_copy(
            pairs_hbm.at[pl.ds(0, 64), :],
            pbuf.at[buf, :, pl.ds(0, 128)],
            wsems[buf],
        ).wait()

    def transpose(buf, tcols):
        tb = tbuf.at[buf]
        pb = pbuf.at[buf]

        @pl.loop(0, tcols // 16)
        def _(k):
            rv = (it + 16 * k) // 2
            for c0 in range(0, 64, 16):
                vs = [tb[c0 + d, pl.ds(16 * k, 16)] for d in range(16)]
                for d in range(16):
                    plsc.store_scatter(pb, [rv, par64 + (c0 + d)], vs[d])

    def step_a(b, buf, jjd):
        @pl.when(b < NFULL)
        def _():
            drain(buf)
            fire(b + NW, 1 - buf)

            @pl.when(jjd >= 2)
            def _():
                drain_w(buf)

            transpose(buf, BLK)
            pltpu.async_copy(
                pbuf.at[buf, :, pl.ds(0, 128)],
                pairs_hbm.at[pl.ds(b * 64, 64), :],
                wsems[buf],
            )

    fire(wid, 0)

    @pl.loop(0, 24, step=2)
    def _(jj):
        for d in range(2):
            step_a(wid + NW * (jj + d), d, jj + d)

    step_a(wid + NW * 24, 0, 24)
    drain_w(0)
    drain_w(1)

    @pl.when(wid == NW - 1)
    def _():
        pltpu.sync_copy(tail_hbm, tbuf.at[0])
        transpose(0, TAIL)
        pltpu.sync_copy(
            pbuf.at[0, pl.ds(0, 16), pl.ds(0, 128)],
            pairs_hbm.at[pl.ds(NFULL * 64, 16), :],
        )


@functools.partial(
    pl.kernel,
    mesh=_mesh,
    compiler_params=_params_b,
    out_type=jax.ShapeDtypeStruct((EMB_DIM, N_NODES), jnp.float32),
    scratch_types=[
        pltpu.VMEM((BLK,), jnp.int32),
        pltpu.VMEM((2, BLK), jnp.int32),
        pltpu.VMEM((BLK,), jnp.int32),
        pltpu.VMEM((2, BLK, 129), jnp.float32),
        pltpu.VMEM((2, 64, BLK), jnp.float32),
        pltpu.SemaphoreType.DMA,
        pltpu.SemaphoreType.DMA,
        pltpu.SemaphoreType.DMA,
        pltpu.SemaphoreType.DMA,
    ],
)
def _gather_kernel(
    idx_hbm, pairs_hbm, outt_hbm, icb, hob, ipb, gbuf, obuf, s0, s1, w0, w1
):
    wid = lax.axis_index("s") * NUM_CORES + lax.axis_index("c")
    it = _iota()
    ridx = [it + 16 * k for k in range(8)]
    zero = it * 0
    sems = [s0, s1]
    wsems = [w0, w1]

    def fire(b, buf):
        @pl.when(b < NBLK_B)
        def _():
            pltpu.sync_copy(idx_hbm.at[pl.ds(b * BLK, BLK)], icb)
            for k in range(8):
                v = icb[pl.ds(16 * k, 16)]
                ipb[pl.ds(16 * k, 16)] = v // 2
                hob[buf, pl.ds(16 * k, 16)] = (v % 2) * 64
            pltpu.async_copy(
                pairs_hbm.at[ipb], gbuf.at[buf, :, pl.ds(0, 128)], sems[buf]
            )

    def drain(buf):
        pltpu.make_async_copy(
            pairs_hbm.at[pl.ds(0, BLK), :],
            gbuf.at[buf, :, pl.ds(0, 128)],
            sems[buf],
        ).wait()

    def drain_w(buf):
        pltpu.make_async_copy(
            pairs_hbm.at[pl.ds(0, 64), :], obuf.at[buf], wsems[buf]
        ).wait()

    def step_b(b, buf, jjd):
        @pl.when(b < NBLK_B)
        def _():
            drain(buf)
            fire(b + NW, 1 - buf)

            @pl.when(jjd >= 2)
            def _():
                drain_w(buf)

            gb = gbuf.at[buf]
            ob = obuf.at[buf]

            @pl.loop(0, 8)
            def _(k):
                rv = it + 16 * k
                hvec = hob[buf, pl.ds(16 * k, 16)]
                for c0 in range(0, 64, 16):
                    vs = [
                        plsc.load_gather(gb, [rv, hvec + (c0 + d)])
                        for d in range(16)
                    ]
                    for d in range(16):
                        plsc.store_scatter(ob, [zero + (c0 + d), rv], vs[d])
            pltpu.async_copy(
                ob, outt_hbm.at[:, pl.ds(b * BLK, BLK)], wsems[buf]
            )

    fire(wid, 0)

    @pl.loop(0, 24, step=2)
    def _(jj):
        for d in range(2):
            step_b(wid + NW * (jj + d), d, jj + d)

    step_b(wid + NW * 24, 0, 24)
    drain_w(0)
    drain_w(1)


def kernel(x, table):
    idx = x[:, 0]
    idx_pad = jnp.concatenate([idx, jnp.zeros((N_PAD - N_NODES,), jnp.int32)])
    tail = jnp.pad(table.T[:, NFULL * BLK:], ((0, 0), (0, BLK - TAIL)))
    pairs = _pairs_kernel(table.T, tail)
    outt = _gather_kernel(idx_pad, pairs)
    return outt.T


# speedup vs baseline: 1.4586x; 1.4586x over previous
import functools

import jax
import jax.numpy as jnp
from jax import lax
from jax.experimental import pallas as pl
from jax.experimental.pallas import tpu as pltpu
from jax.experimental.pallas import tpu_sc as plsc

N_NODES = 100000
EMB_DIM = 64

NUM_CORES = 2
NUM_SUBCORES = 16
NW = NUM_CORES * NUM_SUBCORES

CHUNK = 128
ROWS_PER_W = 3128
NCH = 25
LAST_OFF = ROWS_PER_W - CHUNK
NBUF = 4

_mesh = plsc.VectorSubcoreMesh(core_axis_name="c", subcore_axis_name="s")


@functools.partial(
    pl.kernel,
    mesh=_mesh,
    compiler_params=pltpu.CompilerParams(use_tc_tiling_on_sc=False),
    out_type=jax.ShapeDtypeStruct((N_NODES, EMB_DIM), jnp.float32),
    scratch_types=[
        pltpu.VMEM((ROWS_PER_W,), jnp.int32),
        pltpu.VMEM((NBUF, CHUNK, EMB_DIM), jnp.float32),
        pltpu.SemaphoreType.DMA,
        pltpu.SemaphoreType.DMA,
        pltpu.SemaphoreType.DMA,
        pltpu.SemaphoreType.DMA,
    ],
)
def _gather_kernel(idx_hbm, table_hbm, out_hbm, idx_v, rows_v, g0, g1, g2, g3):
    gs = [g0, g1, g2, g3]
    wid = lax.axis_index("s") * NUM_CORES + lax.axis_index("c")
    start = wid * ROWS_PER_W
    base = jnp.minimum(start, N_NODES - ROWS_PER_W)
    pltpu.sync_copy(idx_hbm.at[pl.ds(base, ROWS_PER_W)], idx_v)
    loff = start - base

    def off(c):
        return jnp.minimum(loff + c * CHUNK, LAST_OFF)

    def fire(c, b):
        pltpu.async_copy(
            table_hbm.at[idx_v.at[pl.ds(off(c), CHUNK)]], rows_v.at[b], gs[b]
        )

    def drain(b):
        pltpu.make_async_copy(
            table_hbm.at[pl.ds(0, CHUNK), :], rows_v.at[b], gs[b]
        ).wait()

    def writeout(c, b):
        pltpu.sync_copy(rows_v.at[b], out_hbm.at[pl.ds(base + off(c), CHUNK), :])

    for b in range(NBUF):
        fire(b, b)

    @pl.loop(0, NCH - 1, step=NBUF)
    def _(j):
        for b in range(NBUF):
            c = j + b
            drain(b)
            writeout(c, b)

            @pl.when(c + NBUF < NCH)
            def _():
                fire(c + NBUF, b)

    drain(0)
    writeout(NCH - 1, 0)


def kernel(x, table):
    return _gather_kernel(x[:, 0], table)
